# initial kernel scaffold (unmeasured)
import jax
import jax.numpy as jnp
from jax import lax
from jax.experimental import pallas as pl
from jax.experimental.pallas import tpu as pltpu

T = 512
D = 1024
V_SHARD = 8192
V_CHUNK = 2048


def _body(x_ref, w_ref, labels_ref, out_ref, comm_ref, send_sem, recv_sem):
    my_x = lax.axis_index("x")
    my_y = lax.axis_index("y")
    peer = (my_x, 1 - my_y)

    barrier_sem = pltpu.get_barrier_semaphore()
    pl.semaphore_signal(
        barrier_sem, inc=1, device_id=peer, device_id_type=pl.DeviceIdType.MESH
    )
    pl.semaphore_wait(barrier_sem, 1)

    xb = x_ref[...].astype(jnp.bfloat16)
    labels = labels_ref[...]
    v_base = my_y * V_SHARD

    m = None
    s = None
    lab = None
    for c in range(V_SHARD // V_CHUNK):
        wb = w_ref[:, c * V_CHUNK : (c + 1) * V_CHUNK].astype(jnp.bfloat16)
        logits_t = lax.dot_general(
            wb,
            xb,
            dimension_numbers=(((0,), (1,)), ((), ())),
            preferred_element_type=jnp.float32,
        )
        m_c = jnp.max(logits_t, axis=0, keepdims=True)
        s_c = jnp.sum(jnp.exp(logits_t - m_c), axis=0, keepdims=True)
        loc = labels - (v_base + c * V_CHUNK)
        vio = lax.broadcasted_iota(jnp.int32, (V_CHUNK, T), 0)
        lab_c = jnp.sum(
            jnp.where(vio == loc, logits_t, 0.0), axis=0, keepdims=True
        )
        if m is None:
            m, s, lab = m_c, s_c, lab_c
        else:
            mm = jnp.maximum(m, m_c)
            s = s * jnp.exp(m - mm) + s_c * jnp.exp(m_c - mm)
            m = mm
            lab = lab + lab_c

    comm_ref[0, 0:1, :] = m
    comm_ref[0, 1:2, :] = s
    comm_ref[0, 2:3, :] = lab

    rdma = pltpu.make_async_remote_copy(
        src_ref=comm_ref.at[0],
        dst_ref=comm_ref.at[1],
        send_sem=send_sem,
        recv_sem=recv_sem,
        device_id=peer,
        device_id_type=pl.DeviceIdType.MESH,
    )
    rdma.start()
    rdma.wait()

    m_o = comm_ref[1, 0:1, :]
    s_o = comm_ref[1, 1:2, :]
    lab_o = comm_ref[1, 2:3, :]
    mm = jnp.maximum(m, m_o)
    ss = s * jnp.exp(m - mm) + s_o * jnp.exp(m_o - mm)
    lse = mm + jnp.log(ss)
    out_ref[...] = lse - (lab + lab_o)


def kernel(x, W, labels):
    labels2 = labels.reshape(1, T).astype(jnp.int32)
    out = pl.pallas_call(
        _body,
        out_shape=jax.ShapeDtypeStruct((1, T), jnp.float32),
        in_specs=[
            pl.BlockSpec(memory_space=pltpu.VMEM),
            pl.BlockSpec(memory_space=pltpu.VMEM),
            pl.BlockSpec(memory_space=pltpu.VMEM),
        ],
        out_specs=pl.BlockSpec(memory_space=pltpu.VMEM),
        scratch_shapes=[
            pltpu.VMEM((2, 8, T), jnp.float32),
            pltpu.SemaphoreType.DMA,
            pltpu.SemaphoreType.DMA,
        ],
        compiler_params=pltpu.CompilerParams(collective_id=0),
    )(x, W, labels2)
    return out.reshape(T)


# baseline (device time: 29988 ns/iter reference)
import jax
import jax.numpy as jnp
from jax import lax
from jax.experimental import pallas as pl
from jax.experimental.pallas import tpu as pltpu

T = 512
D = 1024
V_SHARD = 8192
V_CHUNK = 2048


def _body(x_ref, w_ref, labels_ref, out_ref, comm_ref, send_sem, recv_sem):
    my_x = lax.axis_index("x")
    my_y = lax.axis_index("y")
    peer = (my_x, 1 - my_y)

    barrier_sem = pltpu.get_barrier_semaphore()
    pl.semaphore_signal(
        barrier_sem, inc=1, device_id=peer, device_id_type=pl.DeviceIdType.MESH
    )
    pl.semaphore_wait(barrier_sem, 1)

    xb = x_ref[...].astype(jnp.bfloat16)
    labels = labels_ref[...]
    v_base = my_y * V_SHARD

    m = None
    s = None
    lab = None
    for c in range(V_SHARD // V_CHUNK):
        wb = w_ref[:, c * V_CHUNK : (c + 1) * V_CHUNK].astype(jnp.bfloat16)
        logits_t = lax.dot_general(
            wb,
            xb,
            dimension_numbers=(((0,), (1,)), ((), ())),
            preferred_element_type=jnp.float32,
        )
        m_c = jnp.max(logits_t, axis=0, keepdims=True)
        s_c = jnp.sum(jnp.exp(logits_t - m_c), axis=0, keepdims=True)
        loc = labels - (v_base + c * V_CHUNK)
        vio = lax.broadcasted_iota(jnp.int32, (V_CHUNK, T), 0)
        lab_c = jnp.sum(
            jnp.where(vio == loc, logits_t, 0.0), axis=0, keepdims=True
        )
        if m is None:
            m, s, lab = m_c, s_c, lab_c
        else:
            mm = jnp.maximum(m, m_c)
            s = s * jnp.exp(m - mm) + s_c * jnp.exp(m_c - mm)
            m = mm
            lab = lab + lab_c

    comm_ref[0, 0:1, :] = m
    comm_ref[0, 1:2, :] = s
    comm_ref[0, 2:3, :] = lab

    rdma = pltpu.make_async_remote_copy(
        src_ref=comm_ref.at[0],
        dst_ref=comm_ref.at[1],
        send_sem=send_sem,
        recv_sem=recv_sem,
        device_id=peer,
        device_id_type=pl.DeviceIdType.MESH,
    )
    rdma.start()
    rdma.wait()

    m_o = comm_ref[1, 0:1, :]
    s_o = comm_ref[1, 1:2, :]
    lab_o = comm_ref[1, 2:3, :]
    mm = jnp.maximum(m, m_o)
    ss = s * jnp.exp(m - mm) + s_o * jnp.exp(m_o - mm)
    lse = mm + jnp.log(ss)
    out_ref[...] = lse - (lab + lab_o)


def kernel(x, W, labels):
    labels2 = labels.reshape(1, T).astype(jnp.int32)
    out = pl.pallas_call(
        _body,
        out_shape=jax.ShapeDtypeStruct((1, T), jnp.float32),
        in_specs=[
            pl.BlockSpec(memory_space=pltpu.VMEM),
            pl.BlockSpec(memory_space=pltpu.VMEM),
            pl.BlockSpec(memory_space=pltpu.VMEM),
        ],
        out_specs=pl.BlockSpec(memory_space=pltpu.VMEM),
        scratch_shapes=[
            pltpu.VMEM((2, 8, T), jnp.float32),
            pltpu.SemaphoreType.DMA,
            pltpu.SemaphoreType.DMA,
        ],
        compiler_params=pltpu.CompilerParams(
            collective_id=0, vmem_limit_bytes=100 * 1024 * 1024
        ),
    )(x, W, labels2)
    return out.reshape(T)


# device time: 17791 ns/iter; 1.6856x vs baseline; 1.6856x over previous
import jax
import jax.numpy as jnp
from jax import lax
from jax.experimental import pallas as pl
from jax.experimental.pallas import tpu as pltpu

T = 512
D = 1024
V_SHARD = 8192
V_LOCAL = V_SHARD // 2
V_CHUNK = 1024
N_CHUNK = V_LOCAL // V_CHUNK


def _body(
    x_ref,
    labels_ref,
    w_hbm,
    out_ref,
    wbuf,
    send_buf,
    recv_buf,
    dma_sems,
    send_sems,
    recv_sems,
):
    my_x = lax.axis_index("x")
    my_y = lax.axis_index("y")
    peers = (
        (my_x, 1 - my_y),
        (1 - my_x, my_y),
        (1 - my_x, 1 - my_y),
    )

    barrier_sem = pltpu.get_barrier_semaphore()
    for p in peers:
        pl.semaphore_signal(
            barrier_sem, inc=1, device_id=p, device_id_type=pl.DeviceIdType.MESH
        )

    col0 = my_x * V_LOCAL

    def chunk_dma(c, slot):
        return pltpu.make_async_copy(
            w_hbm.at[:, pl.ds(col0 + c * V_CHUNK, V_CHUNK)],
            wbuf.at[slot],
            dma_sems.at[slot],
        )

    chunk_dma(0, 0).start()

    xb = x_ref[...].astype(jnp.bfloat16)
    labels = labels_ref[...]
    v_base = my_y * V_SHARD + col0
    vio = lax.broadcasted_iota(jnp.int32, (V_CHUNK, T), 0)

    s = jnp.zeros((1, T), jnp.float32)
    lab = jnp.zeros((1, T), jnp.float32)
    for c in range(N_CHUNK):
        if c + 1 < N_CHUNK:
            chunk_dma(c + 1, (c + 1) % 2).start()
        chunk_dma(c, c % 2).wait()
        wb = wbuf[c % 2].astype(jnp.bfloat16)
        logits_t = lax.dot_general(
            wb,
            xb,
            dimension_numbers=(((0,), (1,)), ((), ())),
            preferred_element_type=jnp.float32,
        )
        s = s + jnp.sum(jnp.exp(logits_t), axis=0, keepdims=True)
        loc = labels - (v_base + c * V_CHUNK)
        lab = lab + jnp.sum(
            jnp.where(vio == loc, logits_t, 0.0), axis=0, keepdims=True
        )

    send_buf[0:1, :] = s
    send_buf[1:2, :] = lab

    pl.semaphore_wait(barrier_sem, 3)
    rdmas = []
    for k, p in enumerate(peers):
        rdma = pltpu.make_async_remote_copy(
            src_ref=send_buf,
            dst_ref=recv_buf.at[k],
            send_sem=send_sems.at[k],
            recv_sem=recv_sems.at[k],
            device_id=p,
            device_id_type=pl.DeviceIdType.MESH,
        )
        rdma.start()
        rdmas.append(rdma)
    for rdma in rdmas:
        rdma.wait()

    s_tot = s + recv_buf[0, 0:1, :] + recv_buf[1, 0:1, :] + recv_buf[2, 0:1, :]
    lab_tot = (
        lab + recv_buf[0, 1:2, :] + recv_buf[1, 1:2, :] + recv_buf[2, 1:2, :]
    )
    out_ref[...] = jnp.log(s_tot) - lab_tot


def kernel(x, W, labels):
    labels2 = labels.reshape(1, T).astype(jnp.int32)
    out = pl.pallas_call(
        _body,
        out_shape=jax.ShapeDtypeStruct((1, T), jnp.float32),
        in_specs=[
            pl.BlockSpec(memory_space=pltpu.VMEM),
            pl.BlockSpec(memory_space=pltpu.VMEM),
            pl.BlockSpec(memory_space=pltpu.MemorySpace.HBM),
        ],
        out_specs=pl.BlockSpec(memory_space=pltpu.VMEM),
        scratch_shapes=[
            pltpu.VMEM((2, D, V_CHUNK), jnp.float32),
            pltpu.VMEM((8, T), jnp.float32),
            pltpu.VMEM((3, 8, T), jnp.float32),
            pltpu.SemaphoreType.DMA((2,)),
            pltpu.SemaphoreType.DMA((3,)),
            pltpu.SemaphoreType.DMA((3,)),
        ],
        compiler_params=pltpu.CompilerParams(
            collective_id=0, vmem_limit_bytes=100 * 1024 * 1024
        ),
    )(x, labels2, W)
    return out.reshape(T)


# device time: 16991 ns/iter; 1.7649x vs baseline; 1.0471x over previous
import jax
import jax.numpy as jnp
from jax import lax
from jax.experimental import pallas as pl
from jax.experimental.pallas import tpu as pltpu

T = 512
D = 1024
V_SHARD = 8192
V_LOCAL = V_SHARD // 2
V_CHUNK = 1024
N_CHUNK = V_LOCAL // V_CHUNK


def _body(
    labels_ref,
    x_hbm,
    w_hbm,
    out_ref,
    xbuf,
    wbuf,
    send_buf,
    recv_buf,
    x_sem,
    dma_sems,
    send_sems,
    recv_sems,
):
    my_x = lax.axis_index("x")
    my_y = lax.axis_index("y")
    col0 = my_x * V_LOCAL

    x_dma = pltpu.make_async_copy(x_hbm, xbuf, x_sem)
    x_dma.start()
    w_dmas = []
    for c in range(N_CHUNK):
        d = pltpu.make_async_copy(
            w_hbm.at[:, pl.ds(col0 + c * V_CHUNK, V_CHUNK)],
            wbuf.at[c],
            dma_sems.at[c],
        )
        d.start()
        w_dmas.append(d)

    peers = (
        (my_x, 1 - my_y),
        (1 - my_x, my_y),
        (1 - my_x, 1 - my_y),
    )
    barrier_sem = pltpu.get_barrier_semaphore()
    for p in peers:
        pl.semaphore_signal(
            barrier_sem, inc=1, device_id=p, device_id_type=pl.DeviceIdType.MESH
        )

    x_dma.wait()
    xb = xbuf[...].astype(jnp.bfloat16)
    labels = labels_ref[...].reshape(1, T)
    v_base = my_y * V_SHARD + col0
    vio = lax.broadcasted_iota(jnp.int32, (V_CHUNK, T), 0)

    s = jnp.zeros((1, T), jnp.float32)
    lab = jnp.zeros((1, T), jnp.float32)
    for c in range(N_CHUNK):
        w_dmas[c].wait()
        wb = wbuf[c].astype(jnp.bfloat16)
        logits_t = lax.dot_general(
            wb,
            xb,
            dimension_numbers=(((0,), (1,)), ((), ())),
            preferred_element_type=jnp.float32,
        )
        s = s + jnp.sum(jnp.exp(logits_t), axis=0, keepdims=True)
        loc = labels - (v_base + c * V_CHUNK)
        lab = lab + jnp.sum(
            jnp.where(vio == loc, logits_t, 0.0), axis=0, keepdims=True
        )

    send_buf[0:1, :] = s
    send_buf[1:2, :] = lab

    pl.semaphore_wait(barrier_sem, 3)
    rdmas = []
    for k, p in enumerate(peers):
        rdma = pltpu.make_async_remote_copy(
            src_ref=send_buf,
            dst_ref=recv_buf.at[k],
            send_sem=send_sems.at[k],
            recv_sem=recv_sems.at[k],
            device_id=p,
            device_id_type=pl.DeviceIdType.MESH,
        )
        rdma.start()
        rdmas.append(rdma)
    for rdma in rdmas:
        rdma.wait()

    s_tot = s + recv_buf[0, 0:1, :] + recv_buf[1, 0:1, :] + recv_buf[2, 0:1, :]
    lab_tot = (
        lab + recv_buf[0, 1:2, :] + recv_buf[1, 1:2, :] + recv_buf[2, 1:2, :]
    )
    out_ref[...] = (jnp.log(s_tot) - lab_tot)[0]


def kernel(x, W, labels):
    out = pl.pallas_call(
        _body,
        out_shape=jax.ShapeDtypeStruct((T,), jnp.float32),
        in_specs=[
            pl.BlockSpec(memory_space=pltpu.VMEM),
            pl.BlockSpec(memory_space=pltpu.MemorySpace.HBM),
            pl.BlockSpec(memory_space=pltpu.MemorySpace.HBM),
        ],
        out_specs=pl.BlockSpec(memory_space=pltpu.VMEM),
        scratch_shapes=[
            pltpu.VMEM((T, D), jnp.float32),
            pltpu.VMEM((N_CHUNK, D, V_CHUNK), jnp.float32),
            pltpu.VMEM((8, T), jnp.float32),
            pltpu.VMEM((3, 8, T), jnp.float32),
            pltpu.SemaphoreType.DMA,
            pltpu.SemaphoreType.DMA((N_CHUNK,)),
            pltpu.SemaphoreType.DMA((3,)),
            pltpu.SemaphoreType.DMA((3,)),
        ],
        compiler_params=pltpu.CompilerParams(
            collective_id=0, vmem_limit_bytes=100 * 1024 * 1024
        ),
    )(labels, x, W)
    return out
